# baseline (speedup 1.0000x reference)
import functools

import jax
import jax.numpy as jnp
from jax import lax
from jax.experimental import pallas as pl
from jax.experimental.pallas import tpu as pltpu
from jax.experimental.pallas import tpu_sc as plsc

BATCH = 4096
SEQ = 50
VOCAB = 1000
D = 1000
NSHARD = 8
TAIL = D - 128 * (NSHARD - 1)

NUM_WORKERS = 32
BW = BATCH // NUM_WORKERS

_MESH = plsc.VectorSubcoreMesh(core_axis_name="c", subcore_axis_name="s")


@functools.partial(
    pl.kernel,
    out_type=jax.ShapeDtypeStruct((SEQ, D, BATCH), jnp.float32),
    mesh=_MESH,
    scratch_types=[
        pltpu.VMEM((SEQ, BW), jnp.int32),
        pltpu.VMEM((2, BW, 128), jnp.float32),
        pltpu.VMEM((2, 128, BW), jnp.float32),
        pltpu.SemaphoreType.DMA((2,)),
        pltpu.SemaphoreType.DMA((2,)),
    ],
    compiler_params=pltpu.CompilerParams(use_tc_tiling_on_sc=True, needs_layout_passes=False),
)
def _gather_t(xt_hbm, *refs):
    shards = refs[:NSHARD]
    out_hbm = refs[NSHARD]
    idx_v, gbuf, tbuf, sem_g, sem_w = refs[NSHARD + 1:]

    wid = lax.axis_index("s") * 2 + lax.axis_index("c")
    b0 = wid * BW

    def G(s, cb):
        return pltpu.make_async_copy(shards[cb].at[idx_v.at[s]],
                                     gbuf.at[cb % 2], sem_g.at[cb % 2])

    def W(s, cb):
        width = 128 if cb < NSHARD - 1 else TAIL
        return pltpu.make_async_copy(
            tbuf.at[cb % 2].at[pl.ds(0, width)],
            out_hbm.at[s].at[pl.ds(cb * 128, width), pl.ds(b0, BW)],
            sem_w.at[cb % 2])

    def transpose_block(m):
        def col(c, cr):
            iota = lax.iota(jnp.int32, 16)
            cols = lax.broadcast_in_dim(c, (16,), ())
            for j in range(8):
                v = plsc.load_gather(gbuf.at[m], [iota + 16 * j, cols])
                tbuf[m, c, pl.ds(16 * j, 16)] = v
            return cr

        lax.fori_loop(0, 128, col, 0)

    pltpu.sync_copy(xt_hbm.at[:, pl.ds(b0, BW)], idx_v)

    G(0, 0).start()
    G(0, 1).start()

    def step_s(s, cr):
        for cb in range(NSHARD):
            m = cb % 2
            G(s, cb).wait()

            if cb >= 2:
                W(s, cb - 2).wait()
            else:

                @pl.when(s >= 1)
                def _():
                    W(s - 1, cb + NSHARD - 2).wait()

            transpose_block(m)
            W(s, cb).start()

            if cb < NSHARD - 2:
                G(s, cb + 2).start()
            else:

                @pl.when(s + 1 < SEQ)
                def _():
                    G(s + 1, cb - (NSHARD - 2)).start()

        return cr

    lax.fori_loop(0, SEQ, step_s, 0)
    W(SEQ - 1, NSHARD - 2).wait()
    W(SEQ - 1, NSHARD - 1).wait()


def kernel(x, table):
    tp = jnp.pad(table, ((0, 0), (0, NSHARD * 128 - D)))
    shards = tuple(tp[:, c * 128:(c + 1) * 128] for c in range(NSHARD))
    out_t = _gather_t(x.T, *shards)
    return jnp.transpose(out_t, (2, 0, 1))


# speedup vs baseline: 3.4380x; 3.4380x over previous
import functools

import jax
import jax.numpy as jnp
from jax import lax
from jax.experimental import pallas as pl
from jax.experimental.pallas import tpu as pltpu
from jax.experimental.pallas import tpu_sc as plsc

BATCH = 4096
SEQ = 50
SEQ_PAD = 56
VOCAB = 1000
D = 1000
NSHARD = 8
TAIL = D - 128 * (NSHARD - 1)

NUM_WORKERS = 32
NB = BATCH // NUM_WORKERS
NBUF = 2

_MESH = plsc.VectorSubcoreMesh(core_axis_name="c", subcore_axis_name="s")


@functools.partial(
    pl.kernel,
    out_type=jax.ShapeDtypeStruct((BATCH, SEQ, D), jnp.float32),
    mesh=_MESH,
    scratch_types=[
        pltpu.VMEM((NB * SEQ_PAD,), jnp.int32),
        pltpu.VMEM((NBUF, NSHARD, SEQ, 128), jnp.float32),
        pltpu.VMEM((SEQ, TAIL), jnp.float32),
        pltpu.SemaphoreType.DMA((NBUF,)),
        pltpu.SemaphoreType.DMA((NBUF,)),
        pltpu.SemaphoreType.DMA,
    ],
    compiler_params=pltpu.CompilerParams(use_tc_tiling_on_sc=True),
)
def _gather_rows(xf_hbm, *refs):
    shards = refs[:NSHARD]
    out_hbm = refs[NSHARD]
    idx_v, rows, tail_v, sem_g, sem_w, sem_t = refs[NSHARD + 1:]

    wid = lax.axis_index("s") * 2 + lax.axis_index("c")
    base = wid * NB

    def idx_slice(k):
        return idx_v.at[pl.ds(pl.multiple_of(k * SEQ_PAD, 8), SEQ)]

    def gather_copy(k, m, c):
        return pltpu.make_async_copy(shards[c].at[idx_slice(k)],
                                     rows.at[m, c], sem_g.at[m])

    def shard_write(bb, m, c):
        return pltpu.make_async_copy(
            rows.at[m, c], out_hbm.at[bb].at[:, pl.ds(c * 128, 128)],
            sem_w.at[m])

    def tail_write(bb):
        return pltpu.make_async_copy(
            tail_v, out_hbm.at[bb].at[:, pl.ds(128 * (NSHARD - 1), TAIL)],
            sem_t)

    def tail_compact(m):
        def row(s, cr):
            for off in (0, 16, 32, 48, 64, 80, TAIL - 16):
                tail_v[s, pl.ds(off, 16)] = rows[m, NSHARD - 1, s,
                                                 pl.ds(off, 16)]
            return cr

        lax.fori_loop(0, SEQ, row, 0)

    pltpu.sync_copy(xf_hbm.at[pl.ds(base * SEQ_PAD, NB * SEQ_PAD)], idx_v)

    for m in range(NBUF):
        for c in range(NSHARD):
            gather_copy(m, m, c).start()

    def pair(g, cr):
        for m in range(NBUF):
            k = g + m
            bb = base + k
            for c in range(NSHARD):
                gather_copy(k, m, c).wait()
            for c in range(NSHARD - 1):
                shard_write(bb, m, c).start()

            @pl.when(k >= 1)
            def _():
                tail_write(bb - 1).wait()

            tail_compact(m)
            tail_write(bb).start()

            @pl.when(k + NBUF < NB)
            def _():
                for c in range(NSHARD - 1):
                    shard_write(bb, m, c).wait()
                for c in range(NSHARD):
                    gather_copy(k + NBUF, m, c).start()

        return cr

    lax.fori_loop(0, NB // NBUF, lambda i, cr: pair(i * NBUF, cr), 0)

    for m in range(NBUF):
        for c in range(NSHARD - 1):
            shard_write(base + NB - NBUF + m, m, c).wait()
    tail_write(base + NB - 1).wait()


def kernel(x, table):
    x_flat = jnp.pad(x, ((0, 0), (0, SEQ_PAD - SEQ))).reshape(-1)
    tp = jnp.pad(table, ((0, 0), (0, NSHARD * 128 - D)))
    shards = tuple(tp[:, c * 128:(c + 1) * 128] for c in range(NSHARD))
    return _gather_rows(x_flat, *shards)
